# baseline (speedup 1.0000x reference)
import functools

import jax
import jax.numpy as jnp
from jax import lax
from jax.experimental import pallas as pl
from jax.experimental.pallas import tpu as pltpu
from jax.experimental.pallas import tpu_sc as plsc

_ROWS = 16
_COLS = 1024
_D = 64
_NC = 2
_NS = 16
_NW = _NC * _NS
_B_PER_W = _ROWS * _COLS // _NW
_CH = 8
_C = _B_PER_W // _CH

_mesh = plsc.VectorSubcoreMesh(core_axis_name="c", subcore_axis_name="s", num_cores=_NC)


@functools.partial(
    pl.kernel,
    mesh=_mesh,
    compiler_params=pltpu.CompilerParams(use_tc_tiling_on_sc=False),
    out_type=jax.ShapeDtypeStruct((_ROWS, _COLS, _D), jnp.float32),
    scratch_types=[
        pltpu.VMEM((_B_PER_W,), jnp.int32),
        pltpu.VMEM((_B_PER_W, _D), jnp.float32),
    ]
    + [pltpu.SemaphoreType.DMA] * (3 * _CH),
)
def _gather_rows(idx_hbm, table_hbm, out_hbm, idx_v, rows_v, *sems):
    isems, gsems, wsems = sems[:_CH], sems[_CH : 2 * _CH], sems[2 * _CH :]
    wid = lax.axis_index("s") * _NC + lax.axis_index("c")
    per_row = _COLS // _B_PER_W
    row = wid // per_row
    col = (wid % per_row) * _B_PER_W
    iloads = [
        pltpu.async_copy(
            idx_hbm.at[row, pl.ds(col + c * _C, _C)],
            idx_v.at[pl.ds(c * _C, _C)],
            isems[c],
        )
        for c in range(_CH)
    ]
    gathers = []
    for c in range(_CH):
        iloads[c].wait()
        gathers.append(
            pltpu.async_copy(
                table_hbm.at[idx_v.at[pl.ds(c * _C, _C)]],
                rows_v.at[pl.ds(c * _C, _C)],
                gsems[c],
            )
        )
    writes = []
    for c in range(_CH):
        gathers[c].wait()
        writes.append(
            pltpu.async_copy(
                rows_v.at[pl.ds(c * _C, _C)],
                out_hbm.at[row, pl.ds(col + c * _C, _C)],
                wsems[c],
            )
        )
    for w in writes:
        w.wait()


def kernel(embed_id, weight):
    return _gather_rows(embed_id.astype(jnp.int32), weight)


# speedup vs baseline: 1.0169x; 1.0169x over previous
import functools

import jax
import jax.numpy as jnp
from jax import lax
from jax.experimental import pallas as pl
from jax.experimental.pallas import tpu as pltpu
from jax.experimental.pallas import tpu_sc as plsc

_ROWS = 16
_COLS = 1024
_D = 64
_NC = 2
_NS = 16
_NW = _NC * _NS
_B_PER_W = _ROWS * _COLS // _NW
_CH = 4
_C = _B_PER_W // _CH

_mesh = plsc.VectorSubcoreMesh(core_axis_name="c", subcore_axis_name="s", num_cores=_NC)


@functools.partial(
    pl.kernel,
    mesh=_mesh,
    compiler_params=pltpu.CompilerParams(use_tc_tiling_on_sc=False),
    out_type=jax.ShapeDtypeStruct((_ROWS, _COLS, _D), jnp.float32),
    scratch_types=[
        pltpu.VMEM((_B_PER_W,), jnp.int32),
        pltpu.VMEM((_B_PER_W, _D), jnp.float32),
    ]
    + [pltpu.SemaphoreType.DMA] * (2 * _CH),
)
def _gather_rows(idx_hbm, table_hbm, out_hbm, idx_v, rows_v, *sems):
    gsems, wsems = sems[:_CH], sems[_CH:]
    wid = lax.axis_index("s") * _NC + lax.axis_index("c")
    per_row = _COLS // _B_PER_W
    row = wid // per_row
    col = (wid % per_row) * _B_PER_W
    pltpu.sync_copy(idx_hbm.at[row, pl.ds(col, _B_PER_W)], idx_v)
    gathers = [
        pltpu.async_copy(
            table_hbm.at[idx_v.at[pl.ds(c * _C, _C)]],
            rows_v.at[pl.ds(c * _C, _C)],
            gsems[c],
        )
        for c in range(_CH)
    ]
    writes = []
    for c in range(_CH):
        gathers[c].wait()
        writes.append(
            pltpu.async_copy(
                rows_v.at[pl.ds(c * _C, _C)],
                out_hbm.at[row, pl.ds(col + c * _C, _C)],
                wsems[c],
            )
        )
    for w in writes:
        w.wait()


def kernel(embed_id, weight):
    return _gather_rows(embed_id.astype(jnp.int32), weight)
